# Initial kernel scaffold; baseline (speedup 1.0000x reference)
import jax
import jax.numpy as jnp
from jax.experimental import pallas as pl

N = 32768
D = 768
E = 8
K = 2
LOSS_COEF = 0.01
GATE_EPS = 1e-10

BN = 2048


def _cv2(v, n):
    mean = jnp.sum(v) / n
    var = jnp.sum((v - mean) ** 2) / (n - 1)
    return var / (mean * mean + GATE_EPS)


def _moe_kernel(x_ref, wg_ref, bg_ref, we_ref, be_ref,
                out_ref, gs_ref, loss_ref, imp_ref, load_ref):
    i = pl.program_id(0)
    nsteps = pl.num_programs(0)

    xb = x_ref[:, :]
    g = jnp.dot(xb, wg_ref[:, :],
                preferred_element_type=jnp.float32) + bg_ref[0, :]
    eo = jnp.dot(xb, we_ref[:, :],
                 preferred_element_type=jnp.float32) + be_ref[0, :]

    idx = jax.lax.broadcasted_iota(jnp.int32, (BN, E), 1)
    m1 = jnp.max(g, axis=1, keepdims=True)
    a1 = jnp.min(jnp.where(g == m1, idx, E), axis=1, keepdims=True)
    mask1 = idx == a1
    g2 = jnp.where(mask1, -jnp.inf, g)
    m2 = jnp.max(g2, axis=1, keepdims=True)
    a2 = jnp.min(jnp.where(g2 == m2, idx, E), axis=1, keepdims=True)
    mask = mask1 | (idx == a2)

    masked = jnp.where(mask, g, 0.0)
    mx = jnp.max(masked, axis=1, keepdims=True)
    ex = jnp.exp(masked - mx)
    p = ex / jnp.sum(ex, axis=1, keepdims=True)

    gs_ref[:, :] = p
    out_ref[:, :] = jnp.sum(p * eo, axis=1, keepdims=True)

    @pl.when(i == 0)
    def _():
        imp_ref[:, :] = jnp.zeros_like(imp_ref)
        load_ref[:, :] = jnp.zeros_like(load_ref)

    imp_ref[0, :] += jnp.sum(p, axis=0)
    load_ref[0, :] += jnp.sum((p > 0).astype(jnp.float32), axis=0)

    @pl.when(i == nsteps - 1)
    def _():
        imp = imp_ref[0, :]
        load = load_ref[0, :]
        loss_ref[0, 0] = (_cv2(imp, E) + _cv2(load, E)) * LOSS_COEF


@jax.jit
def _run(x, W_gate, b_gate, W_experts, b_experts):
    nsteps = N // BN
    out, gs, loss, _, _ = pl.pallas_call(
        _moe_kernel,
        grid=(nsteps,),
        in_specs=[
            pl.BlockSpec((BN, D), lambda i: (i, 0)),
            pl.BlockSpec((D, E), lambda i: (0, 0)),
            pl.BlockSpec((1, E), lambda i: (0, 0)),
            pl.BlockSpec((D, E), lambda i: (0, 0)),
            pl.BlockSpec((1, E), lambda i: (0, 0)),
        ],
        out_specs=[
            pl.BlockSpec((BN, 1), lambda i: (i, 0)),
            pl.BlockSpec((BN, E), lambda i: (i, 0)),
            pl.BlockSpec((1, 1), lambda i: (0, 0)),
            pl.BlockSpec((1, E), lambda i: (0, 0)),
            pl.BlockSpec((1, E), lambda i: (0, 0)),
        ],
        out_shape=[
            jax.ShapeDtypeStruct((N, 1), jnp.float32),
            jax.ShapeDtypeStruct((N, E), jnp.float32),
            jax.ShapeDtypeStruct((1, 1), jnp.float32),
            jax.ShapeDtypeStruct((1, E), jnp.float32),
            jax.ShapeDtypeStruct((1, E), jnp.float32),
        ],
    )(x, W_gate, b_gate.reshape(1, E), W_experts, b_experts.reshape(1, E))
    return out.reshape(N), loss[0, 0], gs


def kernel(x, W_gate, b_gate, W_experts, b_experts):
    return _run(x, W_gate, b_gate, W_experts, b_experts)


# speedup vs baseline: 3.6894x; 3.6894x over previous
import jax
import jax.numpy as jnp
from jax.experimental import pallas as pl

N = 32768
D = 768
E = 8
K = 2
LOSS_COEF = 0.01
GATE_EPS = 1e-10

BN = 2048


def _cv2(v, n):
    mean = jnp.sum(v) / n
    var = jnp.sum((v - mean) ** 2) / (n - 1)
    return var / (mean * mean + GATE_EPS)


def _moe_kernel(x_ref, wg_ref, bg_ref, we_ref, be_ref,
                out_ref, gs_ref, loss_ref, imp_ref, load_ref):
    i = pl.program_id(0)
    nsteps = pl.num_programs(0)

    xb = x_ref[:, :]
    g = jnp.dot(xb, wg_ref[:, :],
                preferred_element_type=jnp.float32) + bg_ref[0, :]
    eo = jnp.dot(xb, we_ref[:, :],
                 preferred_element_type=jnp.float32) + be_ref[0, :]

    idx = jax.lax.broadcasted_iota(jnp.int32, (BN, E), 1)
    m1 = jnp.max(g, axis=1, keepdims=True)
    a1 = jnp.min(jnp.where(g == m1, idx, E), axis=1, keepdims=True)
    mask1 = idx == a1
    g2 = jnp.where(mask1, -jnp.inf, g)
    m2 = jnp.max(g2, axis=1, keepdims=True)
    a2 = jnp.min(jnp.where(g2 == m2, idx, E), axis=1, keepdims=True)
    mask = mask1 | (idx == a2)

    masked = jnp.where(mask, g, 0.0)
    mx = jnp.max(masked, axis=1, keepdims=True)
    ex = jnp.exp(masked - mx)
    p = ex / jnp.sum(ex, axis=1, keepdims=True)

    gs_ref[:, :] = p
    out_ref[:, :] = jnp.sum(p * eo, axis=1, keepdims=True)

    @pl.when(i == 0)
    def _():
        imp_ref[:, :] = jnp.zeros_like(imp_ref)
        load_ref[:, :] = jnp.zeros_like(load_ref)

    imp_ref[:, :] += jnp.sum(p, axis=0, keepdims=True)
    load_ref[:, :] += jnp.sum((p > 0).astype(jnp.float32), axis=0, keepdims=True)

    @pl.when(i == nsteps - 1)
    def _():
        imp = imp_ref[:, :]
        load = load_ref[:, :]
        loss = (_cv2(imp, E) + _cv2(load, E)) * LOSS_COEF
        loss_ref[:, :] = jnp.full((1, 1), loss, dtype=jnp.float32)


@jax.jit
def _run(x, W_gate, b_gate, W_experts, b_experts):
    nsteps = N // BN
    out, gs, loss, _, _ = pl.pallas_call(
        _moe_kernel,
        grid=(nsteps,),
        in_specs=[
            pl.BlockSpec((BN, D), lambda i: (i, 0)),
            pl.BlockSpec((D, E), lambda i: (0, 0)),
            pl.BlockSpec((1, E), lambda i: (0, 0)),
            pl.BlockSpec((D, E), lambda i: (0, 0)),
            pl.BlockSpec((1, E), lambda i: (0, 0)),
        ],
        out_specs=[
            pl.BlockSpec((BN, 1), lambda i: (i, 0)),
            pl.BlockSpec((BN, E), lambda i: (i, 0)),
            pl.BlockSpec((1, 1), lambda i: (0, 0)),
            pl.BlockSpec((1, E), lambda i: (0, 0)),
            pl.BlockSpec((1, E), lambda i: (0, 0)),
        ],
        out_shape=[
            jax.ShapeDtypeStruct((N, 1), jnp.float32),
            jax.ShapeDtypeStruct((N, E), jnp.float32),
            jax.ShapeDtypeStruct((1, 1), jnp.float32),
            jax.ShapeDtypeStruct((1, E), jnp.float32),
            jax.ShapeDtypeStruct((1, E), jnp.float32),
        ],
    )(x, W_gate, b_gate.reshape(1, E), W_experts, b_experts.reshape(1, E))
    return out.reshape(N), loss[0, 0], gs


def kernel(x, W_gate, b_gate, W_experts, b_experts):
    return _run(x, W_gate, b_gate, W_experts, b_experts)


# speedup vs baseline: 4.1459x; 1.1237x over previous
import functools

import jax
import jax.numpy as jnp
from jax import lax
from jax.experimental import pallas as pl
from jax.experimental.pallas import tpu as pltpu
from jax.experimental.pallas import tpu_sc as plsc

N = 32768
D = 768
E = 8
LOSS_COEF = 0.01
GATE_EPS = 1e-10

BN = 2048
NW = 32
TPW = N // NW
NGROUPS = TPW // 16


def _matmul_kernel(x_ref, wt_ref, bt_ref, geot_ref):
    geot_ref[:, :] = lax.dot_general(
        wt_ref[:, :], x_ref[:, :],
        dimension_numbers=(((1,), (1,)), ((), ())),
        preferred_element_type=jnp.float32,
    ) + bt_ref[:, :]


@jax.jit
def _matmul(x, WcatT, bcatT):
    return pl.pallas_call(
        _matmul_kernel,
        grid=(N // BN,),
        in_specs=[
            pl.BlockSpec((BN, D), lambda i: (i, 0)),
            pl.BlockSpec((2 * E, D), lambda i: (0, 0)),
            pl.BlockSpec((2 * E, 1), lambda i: (0, 0)),
        ],
        out_specs=pl.BlockSpec((2 * E, BN), lambda i: (0, i)),
        out_shape=jax.ShapeDtypeStruct((2 * E, N), jnp.float32),
    )(x, WcatT, bcatT)


def _routing_kernel(geot_hbm, out_hbm, gst_hbm, imp_hbm, load_hbm,
                    geot_v, pst_v, out_v, stat_v):
    wid = lax.axis_index("s") * 2 + lax.axis_index("c")

    pltpu.sync_copy(geot_hbm.at[:, pl.ds(wid * TPW, TPW)], geot_v)

    zero16 = jnp.zeros((16,), jnp.float32)
    neg_inf = jnp.full((16,), -jnp.inf, jnp.float32)

    def group(t, carry):
        base = t * 16
        g = [geot_v[e, pl.ds(base, 16)] for e in range(E)]
        eo = [geot_v[E + e, pl.ds(base, 16)] for e in range(E)]

        m1 = g[0]
        for e in range(1, E):
            m1 = jnp.maximum(m1, g[e])
        a1 = jnp.full((16,), E, jnp.int32)
        for e in range(E - 1, -1, -1):
            a1 = jnp.where(g[e] == m1, jnp.full((16,), e, jnp.int32), a1)
        g2 = [jnp.where(a1 == e, neg_inf, g[e]) for e in range(E)]
        m2 = g2[0]
        for e in range(1, E):
            m2 = jnp.maximum(m2, g2[e])
        a2 = jnp.full((16,), E, jnp.int32)
        for e in range(E - 1, -1, -1):
            a2 = jnp.where(g2[e] == m2, jnp.full((16,), e, jnp.int32), a2)

        mx = jnp.maximum(m1, zero16)
        ex = []
        s = zero16
        for e in range(E):
            keep = (a1 == e) | (a2 == e)
            me = jnp.where(keep, g[e], zero16)
            x_e = jnp.exp(me - mx)
            ex.append(x_e)
            s = s + x_e
        r = 1.0 / s

        acc = zero16
        new_carry = []
        for e in range(E):
            p_e = ex[e] * r
            pst_v[e, pl.ds(base, 16)] = p_e
            acc = acc + p_e * eo[e]
            new_carry.append(carry[e] + p_e)
        for e in range(E):
            p_e = ex[e] * r
            new_carry.append(
                carry[E + e] + jnp.where(p_e > 0, 1.0, 0.0).astype(jnp.float32))
        out_v[pl.ds(base, 16)] = acc
        return tuple(new_carry)

    init = tuple(jnp.zeros((16,), jnp.float32) for _ in range(2 * E))
    stats = lax.fori_loop(0, NGROUPS, group, init)

    for e in range(E):
        stat_v[pl.ds(e * 16, 16)] = stats[e]
        stat_v[pl.ds(128 + e * 16, 16)] = stats[E + e]

    pltpu.sync_copy(out_v, out_hbm.at[pl.ds(wid * TPW, TPW)])
    pltpu.sync_copy(pst_v, gst_hbm.at[:, pl.ds(wid * TPW, TPW)])
    pltpu.sync_copy(stat_v.at[pl.ds(0, 128)], imp_hbm.at[wid])
    pltpu.sync_copy(stat_v.at[pl.ds(128, 128)], load_hbm.at[wid])


@jax.jit
def _routing(geot):
    f = functools.partial(
        pl.kernel,
        out_type=[
            jax.ShapeDtypeStruct((N,), jnp.float32),
            jax.ShapeDtypeStruct((E, N), jnp.float32),
            jax.ShapeDtypeStruct((NW, 128), jnp.float32),
            jax.ShapeDtypeStruct((NW, 128), jnp.float32),
        ],
        mesh=plsc.VectorSubcoreMesh(core_axis_name="c", subcore_axis_name="s"),
        scratch_types=[
            pltpu.VMEM((2 * E, TPW), jnp.float32),
            pltpu.VMEM((E, TPW), jnp.float32),
            pltpu.VMEM((TPW,), jnp.float32),
            pltpu.VMEM((2 * 128,), jnp.float32),
        ],
    )(_routing_kernel)
    return f(geot)


def _finalize_kernel(gst_ref, imp_ref, load_ref, gs_ref, loss_ref):
    i = pl.program_id(0)
    gs_ref[:, :] = gst_ref[:, :].T

    @pl.when(i == 0)
    def _():
        i0 = lax.broadcasted_iota(jnp.int32, (128, E), 0)
        i1 = lax.broadcasted_iota(jnp.int32, (128, E), 1)
        seg = (i0 // 16 == i1).astype(jnp.float32)

        def cv2(part_ref):
            a = jnp.sum(part_ref[:, :], axis=0, keepdims=True)
            v = jnp.dot(a, seg, preferred_element_type=jnp.float32)
            mean = jnp.sum(v) / E
            var = jnp.sum((v - mean) ** 2) / (E - 1)
            return var / (mean * mean + GATE_EPS)

        loss = (cv2(imp_ref) + cv2(load_ref)) * LOSS_COEF
        loss_ref[:, :] = jnp.full((1, 1), loss, dtype=jnp.float32)


@jax.jit
def _finalize(gst, impP, loadP):
    return pl.pallas_call(
        _finalize_kernel,
        grid=(N // BN,),
        in_specs=[
            pl.BlockSpec((E, BN), lambda i: (0, i)),
            pl.BlockSpec((NW, 128), lambda i: (0, 0)),
            pl.BlockSpec((NW, 128), lambda i: (0, 0)),
        ],
        out_specs=[
            pl.BlockSpec((BN, E), lambda i: (i, 0)),
            pl.BlockSpec((1, 1), lambda i: (0, 0)),
        ],
        out_shape=[
            jax.ShapeDtypeStruct((N, E), jnp.float32),
            jax.ShapeDtypeStruct((1, 1), jnp.float32),
        ],
    )(gst, impP, loadP)


def kernel(x, W_gate, b_gate, W_experts, b_experts):
    WcatT = jnp.concatenate([W_gate, W_experts], axis=1).T
    bcatT = jnp.concatenate([b_gate, b_experts]).reshape(2 * E, 1)
    geot = _matmul(x, WcatT, bcatT)
    out, gst, impP, loadP = _routing(geot)
    gs, loss = _finalize(gst, impP, loadP)
    return out, loss[0, 0], gs


# speedup vs baseline: 4.3688x; 1.0538x over previous
import functools

import jax
import jax.numpy as jnp
from jax import lax
from jax.experimental import pallas as pl
from jax.experimental.pallas import tpu as pltpu
from jax.experimental.pallas import tpu_sc as plsc

N = 32768
D = 768
E = 8
LOSS_COEF = 0.01
GATE_EPS = 1e-10

BN = 4096
NW = 32
TPW = N // NW
NGROUPS = TPW // 16


def _matmul_kernel(x_ref, wt_ref, bt_ref, geot_ref):
    geot_ref[:, :] = lax.dot_general(
        wt_ref[:, :], x_ref[:, :],
        dimension_numbers=(((1,), (1,)), ((), ())),
        preferred_element_type=jnp.float32,
    ) + bt_ref[:, :]


@jax.jit
def _matmul(x, WcatT, bcatT):
    return pl.pallas_call(
        _matmul_kernel,
        grid=(N // BN,),
        in_specs=[
            pl.BlockSpec((BN, D), lambda i: (i, 0)),
            pl.BlockSpec((2 * E, D), lambda i: (0, 0)),
            pl.BlockSpec((2 * E, 1), lambda i: (0, 0)),
        ],
        out_specs=pl.BlockSpec((2 * E, BN), lambda i: (0, i)),
        out_shape=jax.ShapeDtypeStruct((2 * E, N), jnp.float32),
    )(x, WcatT, bcatT)


def _routing_kernel(geot_hbm, out_hbm, gst_hbm, imp_hbm, load_hbm,
                    geot_v, pst_v, out_v, stat_v):
    wid = lax.axis_index("s") * 2 + lax.axis_index("c")

    pltpu.sync_copy(geot_hbm.at[:, pl.ds(wid * TPW, TPW)], geot_v)

    zero16 = jnp.zeros((16,), jnp.float32)
    neg_inf = jnp.full((16,), -jnp.inf, jnp.float32)

    def group(t, carry):
        base = t * 16
        g = [geot_v[e, pl.ds(base, 16)] for e in range(E)]
        eo = [geot_v[E + e, pl.ds(base, 16)] for e in range(E)]

        m1 = g[0]
        for e in range(1, E):
            m1 = jnp.maximum(m1, g[e])
        a1 = jnp.full((16,), E, jnp.int32)
        for e in range(E - 1, -1, -1):
            a1 = jnp.where(g[e] == m1, jnp.full((16,), e, jnp.int32), a1)
        g2 = [jnp.where(a1 == e, neg_inf, g[e]) for e in range(E)]
        m2 = g2[0]
        for e in range(1, E):
            m2 = jnp.maximum(m2, g2[e])
        a2 = jnp.full((16,), E, jnp.int32)
        for e in range(E - 1, -1, -1):
            a2 = jnp.where(g2[e] == m2, jnp.full((16,), e, jnp.int32), a2)

        mx = jnp.maximum(m1, zero16)
        ex = []
        s = zero16
        for e in range(E):
            keep = (a1 == e) | (a2 == e)
            me = jnp.where(keep, g[e], zero16)
            x_e = jnp.exp(me - mx)
            ex.append(x_e)
            s = s + x_e
        r = 1.0 / s

        acc = zero16
        new_carry = []
        for e in range(E):
            p_e = ex[e] * r
            pst_v[e, pl.ds(base, 16)] = p_e
            acc = acc + p_e * eo[e]
            new_carry.append(carry[e] + p_e)
        for e in range(E):
            p_e = ex[e] * r
            new_carry.append(
                carry[E + e] + jnp.where(p_e > 0, 1.0, 0.0).astype(jnp.float32))
        out_v[pl.ds(base, 16)] = acc
        return tuple(new_carry)

    init = tuple(jnp.zeros((16,), jnp.float32) for _ in range(2 * E))
    stats = lax.fori_loop(0, NGROUPS, group, init)

    for e in range(E):
        stat_v[pl.ds(e * 16, 16)] = stats[e]
        stat_v[pl.ds(128 + e * 16, 16)] = stats[E + e]

    pltpu.sync_copy(out_v, out_hbm.at[pl.ds(wid * TPW, TPW)])
    pltpu.sync_copy(pst_v, gst_hbm.at[:, pl.ds(wid * TPW, TPW)])
    pltpu.sync_copy(stat_v.at[pl.ds(0, 128)], imp_hbm.at[wid])
    pltpu.sync_copy(stat_v.at[pl.ds(128, 128)], load_hbm.at[wid])


@jax.jit
def _routing(geot):
    f = functools.partial(
        pl.kernel,
        out_type=[
            jax.ShapeDtypeStruct((N,), jnp.float32),
            jax.ShapeDtypeStruct((E, N), jnp.float32),
            jax.ShapeDtypeStruct((NW, 128), jnp.float32),
            jax.ShapeDtypeStruct((NW, 128), jnp.float32),
        ],
        mesh=plsc.VectorSubcoreMesh(core_axis_name="c", subcore_axis_name="s"),
        scratch_types=[
            pltpu.VMEM((2 * E, TPW), jnp.float32),
            pltpu.VMEM((E, TPW), jnp.float32),
            pltpu.VMEM((TPW,), jnp.float32),
            pltpu.VMEM((2 * 128,), jnp.float32),
        ],
    )(_routing_kernel)
    return f(geot)


def _finalize_kernel(gst_ref, imp_ref, load_ref, gs_ref, loss_ref):
    i = pl.program_id(0)
    gs_ref[:, :] = gst_ref[:, :].T

    @pl.when(i == 0)
    def _():
        i0 = lax.broadcasted_iota(jnp.int32, (128, E), 0)
        i1 = lax.broadcasted_iota(jnp.int32, (128, E), 1)
        seg = (i0 // 16 == i1).astype(jnp.float32)

        def cv2(part_ref):
            a = jnp.sum(part_ref[:, :], axis=0, keepdims=True)
            v = jnp.dot(a, seg, preferred_element_type=jnp.float32)
            mean = jnp.sum(v) / E
            var = jnp.sum((v - mean) ** 2) / (E - 1)
            return var / (mean * mean + GATE_EPS)

        loss = (cv2(imp_ref) + cv2(load_ref)) * LOSS_COEF
        loss_ref[:, :] = jnp.full((1, 1), loss, dtype=jnp.float32)


@jax.jit
def _finalize(gst, impP, loadP):
    return pl.pallas_call(
        _finalize_kernel,
        grid=(N // BN,),
        in_specs=[
            pl.BlockSpec((E, BN), lambda i: (0, i)),
            pl.BlockSpec((NW, 128), lambda i: (0, 0)),
            pl.BlockSpec((NW, 128), lambda i: (0, 0)),
        ],
        out_specs=[
            pl.BlockSpec((BN, E), lambda i: (i, 0)),
            pl.BlockSpec((1, 1), lambda i: (0, 0)),
        ],
        out_shape=[
            jax.ShapeDtypeStruct((N, E), jnp.float32),
            jax.ShapeDtypeStruct((1, 1), jnp.float32),
        ],
    )(gst, impP, loadP)


def kernel(x, W_gate, b_gate, W_experts, b_experts):
    WcatT = jnp.concatenate([W_gate, W_experts], axis=1).T
    bcatT = jnp.concatenate([b_gate, b_experts]).reshape(2 * E, 1)
    geot = _matmul(x, WcatT, bcatT)
    out, gst, impP, loadP = _routing(geot)
    gs, loss = _finalize(gst, impP, loadP)
    return out, loss[0, 0], gs
